# Initial kernel scaffold; baseline (speedup 1.0000x reference)
import jax
import jax.numpy as jnp
from jax.experimental import pallas as pl


def kernel(x, edge_idx, w_pair, w_each):
    raise NotImplementedError("write your pallas kernel here")


# speedup vs baseline: 5.9470x; 5.9470x over previous
import functools

import jax
import jax.numpy as jnp
from jax import lax
from jax.experimental import pallas as pl
from jax.experimental.pallas import tpu as pltpu
from jax.experimental.pallas import tpu_sc as plsc

_N, _D, _E = 10000, 128, 320000
_NW = 32
_EPW = _E // _NW
_C = 200
_NCHUNK = _EPW // _C
_G = _D // 16

_mesh = plsc.VectorSubcoreMesh(core_axis_name="c", subcore_axis_name="s")


@functools.partial(
    pl.kernel,
    out_type=jax.ShapeDtypeStruct((_NW, 16), jnp.float32),
    mesh=_mesh,
    scratch_types=[
        pltpu.VMEM((_C,), jnp.int32),
        pltpu.VMEM((_C,), jnp.int32),
        pltpu.VMEM((_C, _D), jnp.float32),
        pltpu.VMEM((_C, _D), jnp.float32),
        pltpu.VMEM((_D,), jnp.float32),
        pltpu.VMEM((16,), jnp.float32),
        pltpu.SemaphoreType.DMA,
    ],
)
def _sc_edge_sum(x_hbm, src_hbm, dst_hbm, w_hbm, out_hbm,
                 sidx, didx, srows, drows, wv, res, sem):
    wid = lax.axis_index("s") * 2 + lax.axis_index("c")
    base_t = wid * _EPW
    pltpu.sync_copy(w_hbm, wv)
    wvs = [wv[pl.ds(g * 16, 16)] for g in range(_G)]

    def chunk(k, accs):
        base = base_t + k * _C
        pltpu.sync_copy(src_hbm.at[pl.ds(base, _C)], sidx)
        pltpu.sync_copy(dst_hbm.at[pl.ds(base, _C)], didx)
        pltpu.async_copy(x_hbm.at[sidx], srows, sem).wait()
        pltpu.async_copy(x_hbm.at[didx], drows, sem).wait()

        def edge(e, a):
            new = []
            for g in range(_G):
                s = srows[e, pl.ds(g * 16, 16)]
                d = drows[e, pl.ds(g * 16, 16)]
                df = s - d
                new.append(a[g] + df * df * wvs[g])
            return tuple(new)

        return lax.fori_loop(0, _C, edge, accs)

    accs = lax.fori_loop(
        0, _NCHUNK, chunk,
        tuple(jnp.zeros((16,), jnp.float32) for _ in range(_G)))
    tot = accs[0]
    for g in range(1, _G):
        tot = tot + accs[g]
    res[...] = tot
    pltpu.sync_copy(res, out_hbm.at[wid])


def _tc_finish_body(part_ref, x_ref, we_ref, out_ref):
    pair = jnp.sum(part_ref[...])
    each = jnp.sum(x_ref[...] * we_ref[...])
    out_ref[...] = jnp.reshape(pair + each, (1, 1))


def _tc_finish(partials, xf, we2d):
    return pl.pallas_call(
        _tc_finish_body,
        out_shape=jax.ShapeDtypeStruct((1, 1), jnp.float32),
    )(partials, xf, we2d)


def kernel(x, edge_idx, w_pair, w_each):
    b, n, d = x.shape
    xf = x.reshape(n, d)
    partials = _sc_edge_sum(xf, edge_idx[0], edge_idx[1], w_pair.reshape(d))
    out = _tc_finish(partials, xf, w_each.reshape(1, d))
    return out.reshape(b)


# speedup vs baseline: 12.0212x; 2.0214x over previous
import functools

import jax
import jax.numpy as jnp
from jax import lax
from jax.experimental import pallas as pl
from jax.experimental.pallas import tpu as pltpu
from jax.experimental.pallas import tpu_sc as plsc

_N, _D, _E = 10000, 128, 320000
_NW = 32
_EPW = _E // _NW
_C = 200
_NCHUNK = _EPW // _C
_G = _D // 16

_mesh = plsc.VectorSubcoreMesh(core_axis_name="c", subcore_axis_name="s")


@functools.partial(
    pl.kernel,
    out_type=jax.ShapeDtypeStruct((_NW, 16), jnp.float32),
    mesh=_mesh,
    scratch_types=[
        pltpu.VMEM((_EPW,), jnp.int32),
        pltpu.VMEM((_EPW,), jnp.int32),
        pltpu.VMEM((_C, _D), jnp.float32),
        pltpu.VMEM((_C, _D), jnp.float32),
        pltpu.VMEM((_C, _D), jnp.float32),
        pltpu.VMEM((_C, _D), jnp.float32),
        pltpu.VMEM((_D,), jnp.float32),
        pltpu.VMEM((16,), jnp.float32),
        pltpu.SemaphoreType.DMA,
        pltpu.SemaphoreType.DMA,
        pltpu.SemaphoreType.DMA,
        pltpu.SemaphoreType.DMA,
    ],
)
def _sc_edge_sum(x_hbm, src_hbm, dst_hbm, w_hbm, out_hbm,
                 sidx, didx, sr0, dr0, sr1, dr1, wv, res,
                 ss0, sd0, ss1, sd1):
    wid = lax.axis_index("s") * 2 + lax.axis_index("c")
    base_t = wid * _EPW
    pltpu.sync_copy(src_hbm.at[pl.ds(base_t, _EPW)], sidx)
    pltpu.sync_copy(dst_hbm.at[pl.ds(base_t, _EPW)], didx)
    pltpu.sync_copy(w_hbm, wv)
    wvs = [wv[pl.ds(g * 16, 16)] for g in range(_G)]
    srows, drows = [sr0, sr1], [dr0, dr1]
    ssem, dsem = [ss0, ss1], [sd0, sd1]

    def start(cid, b):
        pltpu.async_copy(x_hbm.at[sidx.at[pl.ds(cid * _C, _C)]], srows[b], ssem[b])
        pltpu.async_copy(x_hbm.at[didx.at[pl.ds(cid * _C, _C)]], drows[b], dsem[b])

    start(0, 0)
    start(1, 1)

    def outer(k, accs):
        for b in range(2):
            cid = k * 2 + b
            pltpu.make_async_copy(x_hbm.at[pl.ds(0, _C)], srows[b], ssem[b]).wait()
            pltpu.make_async_copy(x_hbm.at[pl.ds(0, _C)], drows[b], dsem[b]).wait()
            sr, dr = srows[b], drows[b]

            def edge(e, a, sr=sr, dr=dr):
                new = []
                for g in range(_G):
                    s = sr[e, pl.ds(g * 16, 16)]
                    d = dr[e, pl.ds(g * 16, 16)]
                    df = s - d
                    new.append(a[g] + df * df * wvs[g])
                return tuple(new)

            accs = lax.fori_loop(0, _C, edge, accs)

            @pl.when(cid + 2 < _NCHUNK)
            def _():
                start(cid + 2, b)
        return accs

    accs = lax.fori_loop(
        0, _NCHUNK // 2, outer,
        tuple(jnp.zeros((16,), jnp.float32) for _ in range(_G)))
    tot = accs[0]
    for g in range(1, _G):
        tot = tot + accs[g]
    res[...] = tot
    pltpu.sync_copy(res, out_hbm.at[wid])


def _tc_finish_body(part_ref, x_ref, we_ref, out_ref):
    pair = jnp.sum(part_ref[...])
    each = jnp.sum(x_ref[...] * we_ref[...])
    out_ref[...] = jnp.reshape(pair + each, (1, 1))


def _tc_finish(partials, xf, we2d):
    return pl.pallas_call(
        _tc_finish_body,
        out_shape=jax.ShapeDtypeStruct((1, 1), jnp.float32),
    )(partials, xf, we2d)


def kernel(x, edge_idx, w_pair, w_each):
    b, n, d = x.shape
    xf = x.reshape(n, d)
    partials = _sc_edge_sum(xf, edge_idx[0], edge_idx[1], w_pair.reshape(d))
    out = _tc_finish(partials, xf, w_each.reshape(1, d))
    return out.reshape(b)


# speedup vs baseline: 13.6073x; 1.1319x over previous
import functools

import jax
import jax.numpy as jnp
from jax import lax
from jax.experimental import pallas as pl
from jax.experimental.pallas import tpu as pltpu
from jax.experimental.pallas import tpu_sc as plsc

_N, _D, _E = 10000, 128, 320000
_NW = 32
_EPW = _E // _NW
_C = 200
_NCHUNK = _EPW // _C
_G = _D // 16
_W = _D // 2

_mesh = plsc.VectorSubcoreMesh(core_axis_name="c", subcore_axis_name="s")


@functools.partial(
    pl.kernel,
    out_type=jax.ShapeDtypeStruct((_NW, 16), jnp.float32),
    mesh=_mesh,
    scratch_types=[
        pltpu.VMEM((_EPW,), jnp.int32),
        pltpu.VMEM((_EPW,), jnp.int32),
        pltpu.VMEM((_C, _W), jnp.int32),
        pltpu.VMEM((_C, _W), jnp.int32),
        pltpu.VMEM((_C, _W), jnp.int32),
        pltpu.VMEM((_C, _W), jnp.int32),
        pltpu.VMEM((_D,), jnp.float32),
        pltpu.VMEM((16,), jnp.float32),
        pltpu.SemaphoreType.DMA,
        pltpu.SemaphoreType.DMA,
        pltpu.SemaphoreType.DMA,
        pltpu.SemaphoreType.DMA,
    ],
    compiler_params=pltpu.CompilerParams(needs_layout_passes=False, use_tc_tiling_on_sc=False),
)
def _sc_edge_sum(x_hbm, src_hbm, dst_hbm, w_hbm, out_hbm,
                 sidx, didx, sr0, dr0, sr1, dr1, wv, res,
                 ss0, sd0, ss1, sd1):
    wid = lax.axis_index("s") * 2 + lax.axis_index("c")
    base_t = wid * _EPW
    pltpu.sync_copy(src_hbm.at[pl.ds(base_t, _EPW)], sidx)
    pltpu.sync_copy(dst_hbm.at[pl.ds(base_t, _EPW)], didx)
    pltpu.sync_copy(w_hbm, wv)
    wvs = [wv[pl.ds(g * 16, 16)] for g in range(_G)]
    srows, drows = [sr0, sr1], [dr0, dr1]
    ssem, dsem = [ss0, ss1], [sd0, sd1]

    def start(cid, b):
        pltpu.async_copy(x_hbm.at[sidx.at[pl.ds(cid * _C, _C)]], srows[b], ssem[b])
        pltpu.async_copy(x_hbm.at[didx.at[pl.ds(cid * _C, _C)]], drows[b], dsem[b])

    start(0, 0)
    start(1, 1)

    def outer(k, accs):
        for b in range(2):
            cid = k * 2 + b
            pltpu.make_async_copy(x_hbm.at[pl.ds(0, _C)], srows[b], ssem[b]).wait()
            pltpu.make_async_copy(x_hbm.at[pl.ds(0, _C)], drows[b], dsem[b]).wait()
            sr, dr = srows[b], drows[b]

            def edge(e, a, sr=sr, dr=dr):
                new = []
                for j in range(_G // 2):
                    sbf = plsc.bitcast(sr[e, pl.ds(j * 16, 16)], jnp.bfloat16)
                    dbf = plsc.bitcast(dr[e, pl.ds(j * 16, 16)], jnp.bfloat16)
                    df = sbf - dbf
                    sq = df * df
                    e0, e1 = plsc.unpack(sq, format=plsc.PackFormat.INTERLEAVED)
                    new.append(a[2 * j] + e0 * wvs[2 * j])
                    new.append(a[2 * j + 1] + e1 * wvs[2 * j + 1])
                return tuple(new)

            accs = lax.fori_loop(0, _C, edge, accs)

            @pl.when(cid + 2 < _NCHUNK)
            def _():
                start(cid + 2, b)
        return accs

    accs = lax.fori_loop(
        0, _NCHUNK // 2, outer,
        tuple(jnp.zeros((16,), jnp.float32) for _ in range(_G)))
    tot = accs[0]
    for g in range(1, _G):
        tot = tot + accs[g]
    res[...] = tot
    pltpu.sync_copy(res, out_hbm.at[wid])


def _tc_finish_body(part_ref, x_ref, we_ref, out_ref):
    pair = jnp.sum(part_ref[...])
    each = jnp.sum(x_ref[...] * we_ref[...])
    out_ref[...] = jnp.reshape(pair + each, (1, 1))


def _tc_finish(partials, xf, we2d):
    return pl.pallas_call(
        _tc_finish_body,
        out_shape=jax.ShapeDtypeStruct((1, 1), jnp.float32),
    )(partials, xf, we2d)


def kernel(x, edge_idx, w_pair, w_each):
    b, n, d = x.shape
    xf = x.reshape(n, d)
    xbf_i32 = jax.lax.bitcast_convert_type(
        xf.astype(jnp.bfloat16).reshape(n, d // 2, 2), jnp.int32)
    w_perm = jnp.transpose(
        w_pair.reshape(d // 32, 16, 2), (0, 2, 1)).reshape(d)
    partials = _sc_edge_sum(xbf_i32, edge_idx[0], edge_idx[1], w_perm)
    out = _tc_finish(partials, xf, w_each.reshape(1, d))
    return out.reshape(b)


# speedup vs baseline: 17.4878x; 1.2852x over previous
import functools

import jax
import jax.numpy as jnp
from jax import lax
from jax.experimental import pallas as pl
from jax.experimental.pallas import tpu as pltpu
from jax.experimental.pallas import tpu_sc as plsc

_N, _D, _E = 10000, 128, 320000
_NW = 32
_EPW = _E // _NW
_C = 200
_NCHUNK = _EPW // _C
_G = _D // 16
_W = _D // 2
_RPT = 320

_mesh = plsc.VectorSubcoreMesh(core_axis_name="c", subcore_axis_name="s")
_params = pltpu.CompilerParams(
    needs_layout_passes=False, use_tc_tiling_on_sc=False)


@functools.partial(
    pl.kernel,
    out_type=jax.ShapeDtypeStruct((_N, _W), jnp.int32),
    mesh=_mesh,
    scratch_types=[
        pltpu.VMEM((_RPT, _D), jnp.float32),
        pltpu.VMEM((_RPT, _W), jnp.int32),
    ],
    compiler_params=_params,
)
def _sc_pack(x_hbm, pk_hbm, rows, pk):
    wid = lax.axis_index("s") * 2 + lax.axis_index("c")
    base = wid * _RPT

    def pack_rows(r, _):
        for j in range(_G // 2):
            a = rows[r, pl.ds(j * 32, 16)]
            b = rows[r, pl.ds(j * 32 + 16, 16)]
            ab = plsc.pack(a, b, format=plsc.PackFormat.INTERLEAVED)
            pk[r, pl.ds(j * 16, 16)] = plsc.bitcast(ab, jnp.int32)
        return 0

    @pl.when(wid < _NW - 1)
    def _():
        pltpu.sync_copy(x_hbm.at[pl.ds(base, _RPT)], rows)
        lax.fori_loop(0, _RPT, pack_rows, 0)
        pltpu.sync_copy(pk, pk_hbm.at[pl.ds(base, _RPT)])

    @pl.when(wid == _NW - 1)
    def _():
        tail = _N - (_NW - 1) * _RPT
        pltpu.sync_copy(x_hbm.at[pl.ds(base, tail)], rows.at[pl.ds(0, tail)])
        lax.fori_loop(0, tail, pack_rows, 0)
        pltpu.sync_copy(pk.at[pl.ds(0, tail)], pk_hbm.at[pl.ds(base, tail)])


@functools.partial(
    pl.kernel,
    out_type=jax.ShapeDtypeStruct((_NW, 16), jnp.float32),
    mesh=_mesh,
    scratch_types=[
        pltpu.VMEM((_EPW,), jnp.int32),
        pltpu.VMEM((_EPW,), jnp.int32),
        pltpu.VMEM((_C, _W), jnp.int32),
        pltpu.VMEM((_C, _W), jnp.int32),
        pltpu.VMEM((_C, _W), jnp.int32),
        pltpu.VMEM((_C, _W), jnp.int32),
        pltpu.VMEM((_D,), jnp.float32),
        pltpu.VMEM((16,), jnp.float32),
        pltpu.SemaphoreType.DMA,
        pltpu.SemaphoreType.DMA,
        pltpu.SemaphoreType.DMA,
        pltpu.SemaphoreType.DMA,
    ],
    compiler_params=_params,
)
def _sc_edge_sum(x_hbm, eidx_hbm, w_hbm, out_hbm,
                 sidx, didx, sr0, dr0, sr1, dr1, wv, res,
                 ss0, sd0, ss1, sd1):
    wid = lax.axis_index("s") * 2 + lax.axis_index("c")
    base_t = wid * _EPW
    pltpu.sync_copy(eidx_hbm.at[0, pl.ds(base_t, _EPW)], sidx)
    pltpu.sync_copy(eidx_hbm.at[1, pl.ds(base_t, _EPW)], didx)
    pltpu.sync_copy(w_hbm, wv)
    wvs = [wv[pl.ds(g * 16, 16)] for g in range(_G)]
    srows, drows = [sr0, sr1], [dr0, dr1]
    ssem, dsem = [ss0, ss1], [sd0, sd1]

    def start(cid, b):
        pltpu.async_copy(x_hbm.at[sidx.at[pl.ds(cid * _C, _C)]], srows[b], ssem[b])
        pltpu.async_copy(x_hbm.at[didx.at[pl.ds(cid * _C, _C)]], drows[b], dsem[b])

    start(0, 0)
    start(1, 1)

    def outer(k, accs):
        for b in range(2):
            cid = k * 2 + b
            pltpu.make_async_copy(x_hbm.at[pl.ds(0, _C)], srows[b], ssem[b]).wait()
            pltpu.make_async_copy(x_hbm.at[pl.ds(0, _C)], drows[b], dsem[b]).wait()
            sr, dr = srows[b], drows[b]

            def edge2(e, a, sr=sr, dr=dr):
                new = list(a)
                for u in range(2):
                    for j in range(_G // 2):
                        si = sr[2 * e + u, pl.ds(j * 16, 16)]
                        di = dr[2 * e + u, pl.ds(j * 16, 16)]
                        df = (plsc.bitcast(si, jnp.bfloat16)
                              - plsc.bitcast(di, jnp.bfloat16))
                        sq = df * df
                        e0, e1 = plsc.unpack(sq, format=plsc.PackFormat.INTERLEAVED)
                        new[2 * j] = new[2 * j] + e0 * wvs[2 * j]
                        new[2 * j + 1] = new[2 * j + 1] + e1 * wvs[2 * j + 1]
                return tuple(new)

            accs = lax.fori_loop(0, _C // 2, edge2, accs)

            @pl.when(cid + 2 < _NCHUNK)
            def _():
                start(cid + 2, b)
        return accs

    accs = lax.fori_loop(
        0, _NCHUNK // 2, outer,
        tuple(jnp.zeros((16,), jnp.float32) for _ in range(_G)))
    tot = accs[0]
    for g in range(1, _G):
        tot = tot + accs[g]
    res[...] = tot
    pltpu.sync_copy(res, out_hbm.at[wid])


def _tc_finish_body(part_ref, x_ref, we_ref, out_ref):
    pair = jnp.sum(part_ref[...])
    each = jnp.sum(x_ref[...] * we_ref[...])
    out_ref[...] = jnp.reshape(pair + each, (1, 1))


def _tc_finish(partials, xf, we2d):
    return pl.pallas_call(
        _tc_finish_body,
        out_shape=jax.ShapeDtypeStruct((1, 1), jnp.float32),
    )(partials, xf, we2d)


def kernel(x, edge_idx, w_pair, w_each):
    b, n, d = x.shape
    xf = x.reshape(n, d)
    xpk = _sc_pack(xf)
    partials = _sc_edge_sum(xpk, edge_idx, w_pair.reshape(d))
    out = _tc_finish(partials, xf, w_each.reshape(1, d))
    return out.reshape(b)


# speedup vs baseline: 18.1542x; 1.0381x over previous
import functools

import jax
import jax.numpy as jnp
from jax import lax
from jax.experimental import pallas as pl
from jax.experimental.pallas import tpu as pltpu
from jax.experimental.pallas import tpu_sc as plsc

_N, _D, _E = 10000, 128, 320000
_NW = 32
_EPW = _E // _NW
_C = 200
_NCHUNK = _EPW // _C
_G = _D // 16
_W = _D // 2
_RPT = 320
_U = 4

_mesh = plsc.VectorSubcoreMesh(core_axis_name="c", subcore_axis_name="s")
_params = pltpu.CompilerParams(
    needs_layout_passes=False, use_tc_tiling_on_sc=False)


@functools.partial(
    pl.kernel,
    out_type=(jax.ShapeDtypeStruct((_N, _W), jnp.int32),
              jax.ShapeDtypeStruct((_NW, 16), jnp.float32)),
    mesh=_mesh,
    scratch_types=[
        pltpu.VMEM((_RPT, _D), jnp.float32),
        pltpu.VMEM((_RPT, _W), jnp.int32),
        pltpu.VMEM((_D,), jnp.float32),
        pltpu.VMEM((_D,), jnp.float32),
        pltpu.VMEM((16,), jnp.float32),
    ],
    compiler_params=_params,
)
def _sc_pack(x_hbm, ws_hbm, we_hbm, pk_hbm, each_hbm, rows, pk, wsv, wev, res):
    wid = lax.axis_index("s") * 2 + lax.axis_index("c")
    base = wid * _RPT
    pltpu.sync_copy(ws_hbm, wsv)
    pltpu.sync_copy(we_hbm, wev)
    ws = [wsv[pl.ds(g * 16, 16)] for g in range(_G)]
    we = [wev[pl.ds(g * 16, 16)] for g in range(_G)]

    def pack_rows(r, acc):
        new = list(acc)
        for j in range(_G // 2):
            a = rows[r, pl.ds(j * 32, 16)]
            b = rows[r, pl.ds(j * 32 + 16, 16)]
            new[2 * j] = new[2 * j] + a * we[2 * j]
            new[2 * j + 1] = new[2 * j + 1] + b * we[2 * j + 1]
            ab = plsc.pack(a * ws[2 * j], b * ws[2 * j + 1],
                           format=plsc.PackFormat.INTERLEAVED)
            pk[r, pl.ds(j * 16, 16)] = plsc.bitcast(ab, jnp.int32)
        return tuple(new)

    zeros = tuple(jnp.zeros((16,), jnp.float32) for _ in range(_G))

    @pl.when(wid < _NW - 1)
    def _():
        pltpu.sync_copy(x_hbm.at[pl.ds(base, _RPT)], rows)
        acc = lax.fori_loop(0, _RPT, pack_rows, zeros)
        pltpu.sync_copy(pk, pk_hbm.at[pl.ds(base, _RPT)])
        tot = acc[0]
        for g in range(1, _G):
            tot = tot + acc[g]
        res[...] = tot
        pltpu.sync_copy(res, each_hbm.at[wid])

    @pl.when(wid == _NW - 1)
    def _():
        tail = _N - (_NW - 1) * _RPT
        pltpu.sync_copy(x_hbm.at[pl.ds(base, tail)], rows.at[pl.ds(0, tail)])
        acc = lax.fori_loop(0, tail, pack_rows, zeros)
        pltpu.sync_copy(pk.at[pl.ds(0, tail)], pk_hbm.at[pl.ds(base, tail)])
        tot = acc[0]
        for g in range(1, _G):
            tot = tot + acc[g]
        res[...] = tot
        pltpu.sync_copy(res, each_hbm.at[wid])


@functools.partial(
    pl.kernel,
    out_type=jax.ShapeDtypeStruct((_NW, 16), jnp.float32),
    mesh=_mesh,
    scratch_types=[
        pltpu.VMEM((_EPW,), jnp.int32),
        pltpu.VMEM((_EPW,), jnp.int32),
        pltpu.VMEM((_C, _W), jnp.int32),
        pltpu.VMEM((_C, _W), jnp.int32),
        pltpu.VMEM((_C, _W), jnp.int32),
        pltpu.VMEM((_C, _W), jnp.int32),
        pltpu.VMEM((_D,), jnp.float32),
        pltpu.VMEM((16,), jnp.float32),
        pltpu.SemaphoreType.DMA,
        pltpu.SemaphoreType.DMA,
        pltpu.SemaphoreType.DMA,
        pltpu.SemaphoreType.DMA,
    ],
    compiler_params=_params,
)
def _sc_edge_sum(x_hbm, eidx_hbm, wsg_hbm, out_hbm,
                 sidx, didx, sr0, dr0, sr1, dr1, wv, res,
                 ss0, sd0, ss1, sd1):
    wid = lax.axis_index("s") * 2 + lax.axis_index("c")
    base_t = wid * _EPW
    pltpu.sync_copy(eidx_hbm.at[0, pl.ds(base_t, _EPW)], sidx)
    pltpu.sync_copy(eidx_hbm.at[1, pl.ds(base_t, _EPW)], didx)
    pltpu.sync_copy(wsg_hbm, wv)
    srows, drows = [sr0, sr1], [dr0, dr1]
    ssem, dsem = [ss0, ss1], [sd0, sd1]

    def start(cid, b):
        pltpu.async_copy(x_hbm.at[sidx.at[pl.ds(cid * _C, _C)]], srows[b], ssem[b])
        pltpu.async_copy(x_hbm.at[didx.at[pl.ds(cid * _C, _C)]], drows[b], dsem[b])

    start(0, 0)
    start(1, 1)

    def outer(k, accs):
        for b in range(2):
            cid = k * 2 + b
            pltpu.make_async_copy(x_hbm.at[pl.ds(0, _C)], srows[b], ssem[b]).wait()
            pltpu.make_async_copy(x_hbm.at[pl.ds(0, _C)], drows[b], dsem[b]).wait()
            sr, dr = srows[b], drows[b]

            def edgeu(e, a, sr=sr, dr=dr):
                new = list(a)
                for u in range(_U):
                    for j in range(_G // 2):
                        si = sr[_U * e + u, pl.ds(j * 16, 16)]
                        di = dr[_U * e + u, pl.ds(j * 16, 16)]
                        df = (plsc.bitcast(si, jnp.bfloat16)
                              - plsc.bitcast(di, jnp.bfloat16))
                        sq = df * df
                        e0, e1 = plsc.unpack(sq, format=plsc.PackFormat.INTERLEAVED)
                        new[2 * j] = new[2 * j] + e0
                        new[2 * j + 1] = new[2 * j + 1] + e1
                return tuple(new)

            accs = lax.fori_loop(0, _C // _U, edgeu, accs)

            @pl.when(cid + 2 < _NCHUNK)
            def _():
                start(cid + 2, b)
        return accs

    accs = lax.fori_loop(
        0, _NCHUNK // 2, outer,
        tuple(jnp.zeros((16,), jnp.float32) for _ in range(_G)))
    tot = accs[0] * wv[pl.ds(0, 16)]
    for g in range(1, _G):
        tot = tot + accs[g] * wv[pl.ds(g * 16, 16)]
    res[...] = tot
    pltpu.sync_copy(res, out_hbm.at[wid])


def _tc_finish_body(pair_ref, each_ref, out_ref):
    tot = jnp.sum(pair_ref[...]) + jnp.sum(each_ref[...])
    out_ref[...] = jnp.reshape(tot, (1, 1))


def _tc_finish(pair_p, each_p):
    return pl.pallas_call(
        _tc_finish_body,
        out_shape=jax.ShapeDtypeStruct((1, 1), jnp.float32),
    )(pair_p, each_p)


def kernel(x, edge_idx, w_pair, w_each):
    b, n, d = x.shape
    xf = x.reshape(n, d)
    wp = w_pair.reshape(d)
    wscale = jnp.sqrt(jnp.abs(wp))
    wsign = jnp.sign(wp)
    xpk, each_p = _sc_pack(xf, wscale, w_each.reshape(d))
    partials = _sc_edge_sum(xpk, edge_idx, wsign)
    out = _tc_finish(partials, each_p)
    return out.reshape(b)


# speedup vs baseline: 20.3440x; 1.1206x over previous
import functools

import jax
import jax.numpy as jnp
from jax import lax
from jax.experimental import pallas as pl
from jax.experimental.pallas import tpu as pltpu
from jax.experimental.pallas import tpu_sc as plsc

_N, _D, _E = 10000, 128, 320000
_NW = 32
_EPW = _E // _NW
_C = 200
_NCHUNK = _EPW // _C
_G = _D // 16
_W = _D // 2
_RPT = 320
_U = 4

_mesh = plsc.VectorSubcoreMesh(core_axis_name="c", subcore_axis_name="s")
_params = pltpu.CompilerParams(
    needs_layout_passes=False, use_tc_tiling_on_sc=False)


@functools.partial(
    pl.kernel,
    out_type=(jax.ShapeDtypeStruct((_N, _W), jnp.int32),
              jax.ShapeDtypeStruct((_NW, 16), jnp.float32)),
    mesh=_mesh,
    scratch_types=[
        pltpu.VMEM((_RPT, _D), jnp.float32),
        pltpu.VMEM((_RPT, _W), jnp.int32),
        pltpu.VMEM((_D,), jnp.float32),
        pltpu.VMEM((_D,), jnp.float32),
        pltpu.VMEM((16,), jnp.float32),
    ],
    compiler_params=_params,
)
def _sc_pack(x_hbm, ws_hbm, we_hbm, pk_hbm, each_hbm, rows, pk, wsv, wev, res):
    wid = lax.axis_index("s") * 2 + lax.axis_index("c")
    base = wid * _RPT
    pltpu.sync_copy(ws_hbm, wsv)
    pltpu.sync_copy(we_hbm, wev)
    ws = [wsv[pl.ds(g * 16, 16)] for g in range(_G)]
    we = [wev[pl.ds(g * 16, 16)] for g in range(_G)]

    def pack_rows(r, acc):
        new = list(acc)
        for j in range(_G // 2):
            a = rows[r, pl.ds(j * 32, 16)]
            b = rows[r, pl.ds(j * 32 + 16, 16)]
            new[2 * j] = new[2 * j] + a * we[2 * j]
            new[2 * j + 1] = new[2 * j + 1] + b * we[2 * j + 1]
            ab = plsc.pack(a * ws[2 * j], b * ws[2 * j + 1],
                           format=plsc.PackFormat.INTERLEAVED)
            pk[r, pl.ds(j * 16, 16)] = plsc.bitcast(ab, jnp.int32)
        return tuple(new)

    zeros = tuple(jnp.zeros((16,), jnp.float32) for _ in range(_G))

    @pl.when(wid < _NW - 1)
    def _():
        pltpu.sync_copy(x_hbm.at[pl.ds(base, _RPT)], rows)
        acc = lax.fori_loop(0, _RPT, pack_rows, zeros)
        pltpu.sync_copy(pk, pk_hbm.at[pl.ds(base, _RPT)])
        tot = acc[0]
        for g in range(1, _G):
            tot = tot + acc[g]
        res[...] = tot
        pltpu.sync_copy(res, each_hbm.at[wid])

    @pl.when(wid == _NW - 1)
    def _():
        tail = _N - (_NW - 1) * _RPT
        pltpu.sync_copy(x_hbm.at[pl.ds(base, tail)], rows.at[pl.ds(0, tail)])
        acc = lax.fori_loop(0, tail, pack_rows, zeros)
        pltpu.sync_copy(pk.at[pl.ds(0, tail)], pk_hbm.at[pl.ds(base, tail)])
        tot = acc[0]
        for g in range(1, _G):
            tot = tot + acc[g]
        res[...] = tot
        pltpu.sync_copy(res, each_hbm.at[wid])


@functools.partial(
    pl.kernel,
    out_type=jax.ShapeDtypeStruct((_NW, 16), jnp.float32),
    mesh=_mesh,
    scratch_types=[
        pltpu.VMEM((_EPW,), jnp.int32),
        pltpu.VMEM((_EPW,), jnp.int32),
        pltpu.VMEM((_C, _W), jnp.int32),
        pltpu.VMEM((_C, _W), jnp.int32),
        pltpu.VMEM((_C, _W), jnp.int32),
        pltpu.VMEM((_C, _W), jnp.int32),
        pltpu.VMEM((_D,), jnp.float32),
        pltpu.VMEM((16,), jnp.float32),
        pltpu.VMEM_SHARED((_N, _W), jnp.int32),
        pltpu.SemaphoreType.DMA,
        pltpu.SemaphoreType.DMA,
        pltpu.SemaphoreType.DMA,
        pltpu.SemaphoreType.DMA,
    ],
    compiler_params=_params,
)
def _sc_edge_sum(x_hbm, eidx_hbm, wsg_hbm, out_hbm,
                 sidx, didx, sr0, dr0, sr1, dr1, wv, res, xsp,
                 ss0, sd0, ss1, sd1):
    wid = lax.axis_index("s") * 2 + lax.axis_index("c")
    sid = lax.axis_index("s")
    base_t = wid * _EPW
    _RS = _N // 16
    pltpu.sync_copy(x_hbm.at[pl.ds(sid * _RS, _RS)], xsp.at[pl.ds(sid * _RS, _RS)])
    pltpu.sync_copy(eidx_hbm.at[0, pl.ds(base_t, _EPW)], sidx)
    pltpu.sync_copy(eidx_hbm.at[1, pl.ds(base_t, _EPW)], didx)
    pltpu.sync_copy(wsg_hbm, wv)
    plsc.subcore_barrier()
    srows, drows = [sr0, sr1], [dr0, dr1]
    ssem, dsem = [ss0, ss1], [sd0, sd1]

    def start(cid, b):
        pltpu.async_copy(x_hbm.at[sidx.at[pl.ds(cid * _C, _C)]], srows[b], ssem[b])
        pltpu.async_copy(xsp.at[didx.at[pl.ds(cid * _C, _C)]], drows[b], dsem[b])

    start(0, 0)
    start(1, 1)

    def outer(k, accs):
        for b in range(2):
            cid = k * 2 + b
            pltpu.make_async_copy(x_hbm.at[pl.ds(0, _C)], srows[b], ssem[b]).wait()
            pltpu.make_async_copy(x_hbm.at[pl.ds(0, _C)], drows[b], dsem[b]).wait()
            sr, dr = srows[b], drows[b]

            def edgeu(e, a, sr=sr, dr=dr):
                new = list(a)
                for u in range(_U):
                    for j in range(_G // 2):
                        si = sr[_U * e + u, pl.ds(j * 16, 16)]
                        di = dr[_U * e + u, pl.ds(j * 16, 16)]
                        df = (plsc.bitcast(si, jnp.bfloat16)
                              - plsc.bitcast(di, jnp.bfloat16))
                        sq = df * df
                        e0, e1 = plsc.unpack(sq, format=plsc.PackFormat.INTERLEAVED)
                        new[2 * j] = new[2 * j] + e0
                        new[2 * j + 1] = new[2 * j + 1] + e1
                return tuple(new)

            accs = lax.fori_loop(0, _C // _U, edgeu, accs)

            @pl.when(cid + 2 < _NCHUNK)
            def _():
                start(cid + 2, b)
        return accs

    accs = lax.fori_loop(
        0, _NCHUNK // 2, outer,
        tuple(jnp.zeros((16,), jnp.float32) for _ in range(_G)))
    tot = accs[0] * wv[pl.ds(0, 16)]
    for g in range(1, _G):
        tot = tot + accs[g] * wv[pl.ds(g * 16, 16)]
    res[...] = tot
    pltpu.sync_copy(res, out_hbm.at[wid])


def _tc_finish_body(pair_ref, each_ref, out_ref):
    tot = jnp.sum(pair_ref[...]) + jnp.sum(each_ref[...])
    out_ref[...] = jnp.reshape(tot, (1, 1))


def _tc_finish(pair_p, each_p):
    return pl.pallas_call(
        _tc_finish_body,
        out_shape=jax.ShapeDtypeStruct((1, 1), jnp.float32),
    )(pair_p, each_p)


def kernel(x, edge_idx, w_pair, w_each):
    b, n, d = x.shape
    xf = x.reshape(n, d)
    wp = w_pair.reshape(d)
    wscale = jnp.sqrt(jnp.abs(wp))
    wsign = jnp.sign(wp)
    xpk, each_p = _sc_pack(xf, wscale, w_each.reshape(d))
    partials = _sc_edge_sum(xpk, edge_idx, wsign)
    out = _tc_finish(partials, each_p)
    return out.reshape(b)
